# baseline (speedup 1.0000x reference)
import functools

import jax
import jax.numpy as jnp
from jax import lax
from jax.experimental import pallas as pl
from jax.experimental.pallas import tpu as pltpu
from jax.experimental.pallas import tpu_sc as plsc

B = 16384
F = 26
V = 100000
D = 32
C = 16

NCHUNK = 7
NWORK = 32
ROWS_PER_WORKER = B // NWORK
WIN_ROWS = 8
NWIN = ROWS_PER_WORKER // WIN_ROWS
WLOOK = WIN_ROWS * F
NG = WLOOK // 16
HALF = WLOOK // 2

MLP_BLOCK = 1024


def _sc_gather(t2, cat_flat, off_c):
    mesh = plsc.VectorSubcoreMesh(core_axis_name="c", subcore_axis_name="s")
    cp = pltpu.CompilerParams(needs_layout_passes=False,
                              use_tc_tiling_on_sc=False)

    @functools.partial(
        pl.kernel,
        mesh=mesh,
        compiler_params=cp,
        out_type=jax.ShapeDtypeStruct((NCHUNK * B, 128), jnp.float32),
        scratch_types=[
            pltpu.VMEM((WLOOK,), jnp.int32),
            pltpu.VMEM((WLOOK,), jnp.int32),
            pltpu.VMEM((WLOOK,), jnp.int32),
            pltpu.VMEM((WLOOK, D), jnp.float32),
            pltpu.VMEM((NCHUNK, WIN_ROWS, 128), jnp.float32),
            pltpu.SemaphoreType.DMA,
        ],
    )
    def gather_kernel(t2_hbm, cat_hbm, off_hbm, out_hbm,
                      cat_v, off_v, tidx_v, rows_v, plane_v, sem):
        wid = lax.axis_index("c") * 16 + lax.axis_index("s")
        pltpu.sync_copy(off_hbm, off_v)

        zeros16 = jnp.full((16,), 0.0, jnp.float32)

        @pl.loop(0, WIN_ROWS)
        def _zero(rb):
            @pl.loop(0, 4)
            def _z(q):
                plane_v[NCHUNK - 1, rb, pl.ds(64 + q * 16, 16)] = zeros16

        @pl.loop(0, NWIN)
        def _window(w):
            row0 = wid * ROWS_PER_WORKER + w * WIN_ROWS
            pos0 = row0 * F
            pltpu.sync_copy(cat_hbm.at[pl.ds(pos0, WLOOK)], cat_v)

            @pl.loop(0, NG)
            def _idx(g):
                sl = pl.ds(g * 16, 16)
                tidx_v[sl] = cat_v[sl] + off_v[sl]

            cp1 = pltpu.async_copy(
                t2_hbm.at[tidx_v.at[pl.ds(0, HALF)]],
                rows_v.at[pl.ds(0, HALF)], sem)
            cp2 = pltpu.async_copy(
                t2_hbm.at[tidx_v.at[pl.ds(HALF, HALF)]],
                rows_v.at[pl.ds(HALF, HALF)], sem)
            cp1.wait()
            cp2.wait()

            @pl.loop(0, WIN_ROWS)
            def _rearrange(rb):
                @pl.loop(0, F)
                def _field(f):
                    r = rb * F + f
                    j = lax.shift_right_logical(f, 2)
                    l = lax.bitwise_and(f, 3) * D
                    plane_v[j, rb, pl.ds(l, 16)] = rows_v[r, pl.ds(0, 16)]
                    plane_v[j, rb, pl.ds(l + 16, 16)] = rows_v[r, pl.ds(16, 16)]

            @pl.loop(0, NCHUNK)
            def _store(j):
                pltpu.sync_copy(
                    plane_v.at[j],
                    out_hbm.at[pl.ds(j * B + row0, WIN_ROWS)])

    return gather_kernel(t2, cat_flat, off_c)


def _mlp_kernel(emb_ref, cont_ref, w0p_ref, w0c_ref, b0_ref, w1_ref, b1_ref,
                w2_ref, b2_ref, out_ref):
    x = jnp.dot(cont_ref[...], w0c_ref[...], preferred_element_type=jnp.float32)
    for j in range(NCHUNK):
        x = x + jnp.dot(emb_ref[j], w0p_ref[j],
                        preferred_element_type=jnp.float32)
    x = jnp.maximum(x + b0_ref[...], 0.0)
    x = jnp.maximum(jnp.dot(x, w1_ref[...], preferred_element_type=jnp.float32)
                    + b1_ref[...], 0.0)
    x = jnp.maximum(jnp.dot(x, w2_ref[...], preferred_element_type=jnp.float32)
                    + b2_ref[...], 0.0)
    out_ref[...] = x


def _mlp(emb3, cont, W0p, W0c, b0, W1, b1, W2, b2):
    grid = (B // MLP_BLOCK,)
    full = lambda shape: pl.BlockSpec(shape, lambda i: tuple(0 for _ in shape))
    return pl.pallas_call(
        _mlp_kernel,
        grid=grid,
        in_specs=[
            pl.BlockSpec((NCHUNK, MLP_BLOCK, 128), lambda i: (0, i, 0)),
            pl.BlockSpec((MLP_BLOCK, C), lambda i: (i, 0)),
            full(W0p.shape), full(W0c.shape), full(b0.shape),
            full(W1.shape), full(b1.shape), full(W2.shape), full(b2.shape),
        ],
        out_specs=pl.BlockSpec((MLP_BLOCK, W2.shape[1]), lambda i: (i, 0)),
        out_shape=jax.ShapeDtypeStruct((B, W2.shape[1]), jnp.float32),
    )(emb3, cont, W0p, W0c, b0, W1, b1, W2, b2)


def kernel(continuous, categorical_indices, tables, W0, b0, W1, b1, W2, b2):
    cat_flat = categorical_indices.reshape(B * F)
    j = jnp.arange(WLOOK, dtype=jnp.int32)
    off_c = (j % F) * V
    emb3 = _sc_gather(tables, cat_flat, off_c).reshape(NCHUNK, B, 128)
    W0e = W0[: F * D]
    W0c = W0[F * D:]
    W0p = jnp.concatenate(
        [W0e, jnp.zeros((NCHUNK * 128 - F * D, W0.shape[1]), W0.dtype)]
    ).reshape(NCHUNK, 128, W0.shape[1])
    return _mlp(emb3, continuous, W0p, W0c, b0[None, :], W1, b1[None, :],
                W2, b2[None, :])


# speedup vs baseline: 1.1060x; 1.1060x over previous
import functools

import jax
import jax.numpy as jnp
from jax import lax
from jax.experimental import pallas as pl
from jax.experimental.pallas import tpu as pltpu
from jax.experimental.pallas import tpu_sc as plsc

B = 16384
F = 26
V = 100000
D = 32
C = 16

NCHUNK = 7
NWORK = 32
PER_WORKER = B * F // NWORK
WLOOK = 512
NWIN = PER_WORKER // WLOOK
NG = WLOOK // 16
NSTREAM = WLOOK // 128

MLP_BLOCK = 1024


def _sc_gather(t2, cat_fmajor):
    mesh = plsc.VectorSubcoreMesh(core_axis_name="c", subcore_axis_name="s")
    cp = pltpu.CompilerParams(needs_layout_passes=False,
                              use_tc_tiling_on_sc=False)

    @functools.partial(
        pl.kernel,
        mesh=mesh,
        compiler_params=cp,
        out_type=jax.ShapeDtypeStruct((NCHUNK * B, 128), jnp.float32),
        scratch_types=[
            pltpu.VMEM((WLOOK,), jnp.int32),
            pltpu.VMEM((WLOOK,), jnp.int32),
            pltpu.VMEM((WLOOK, D), jnp.float32),
            pltpu.SemaphoreType.DMA,
        ],
    )
    def gather_kernel(t2_hbm, cat_hbm, out_hbm, cat_v, tidx_v, rows_v, sem):
        wid = lax.axis_index("c") * 16 + lax.axis_index("s")

        @pl.loop(0, NWIN)
        def _window(w):
            p0 = wid * PER_WORKER + w * WLOOK
            f = lax.shift_right_logical(p0, 14)
            b0 = lax.bitwise_and(p0, B - 1)
            off = f * V
            pltpu.sync_copy(cat_hbm.at[pl.ds(p0, WLOOK)], cat_v)

            @pl.loop(0, NG)
            def _idx(g):
                sl = pl.ds(g * 16, 16)
                tidx_v[sl] = cat_v[sl] + off

            copies = [
                pltpu.async_copy(
                    t2_hbm.at[tidx_v.at[pl.ds(k * 128, 128)]],
                    rows_v.at[pl.ds(k * 128, 128)], sem)
                for k in range(NSTREAM)
            ]
            for c in copies:
                c.wait()

            j = lax.shift_right_logical(f, 2)
            l = lax.bitwise_and(f, 3) * D
            pltpu.sync_copy(
                rows_v,
                out_hbm.at[pl.ds(j * B + b0, WLOOK), pl.ds(l, D)])

            @pl.when(f >= F - 2)
            def _dup():
                pltpu.sync_copy(
                    rows_v,
                    out_hbm.at[pl.ds(j * B + b0, WLOOK), pl.ds(l + 64, D)])

    return gather_kernel(t2, cat_fmajor)


def _mlp_kernel(emb_ref, cont_ref, w0p_ref, w0c_ref, b0_ref, w1_ref, b1_ref,
                w2_ref, b2_ref, out_ref):
    x = jnp.dot(cont_ref[...], w0c_ref[...], preferred_element_type=jnp.float32)
    for j in range(NCHUNK):
        x = x + jnp.dot(emb_ref[j], w0p_ref[j],
                        preferred_element_type=jnp.float32)
    x = jnp.maximum(x + b0_ref[...], 0.0)
    x = jnp.maximum(jnp.dot(x, w1_ref[...], preferred_element_type=jnp.float32)
                    + b1_ref[...], 0.0)
    x = jnp.maximum(jnp.dot(x, w2_ref[...], preferred_element_type=jnp.float32)
                    + b2_ref[...], 0.0)
    out_ref[...] = x


def _mlp(emb3, cont, W0p, W0c, b0, W1, b1, W2, b2):
    grid = (B // MLP_BLOCK,)
    full = lambda shape: pl.BlockSpec(shape, lambda i: tuple(0 for _ in shape))
    return pl.pallas_call(
        _mlp_kernel,
        grid=grid,
        in_specs=[
            pl.BlockSpec((NCHUNK, MLP_BLOCK, 128), lambda i: (0, i, 0)),
            pl.BlockSpec((MLP_BLOCK, C), lambda i: (i, 0)),
            full(W0p.shape), full(W0c.shape), full(b0.shape),
            full(W1.shape), full(b1.shape), full(W2.shape), full(b2.shape),
        ],
        out_specs=pl.BlockSpec((MLP_BLOCK, W2.shape[1]), lambda i: (i, 0)),
        out_shape=jax.ShapeDtypeStruct((B, W2.shape[1]), jnp.float32),
    )(emb3, cont, W0p, W0c, b0, W1, b1, W2, b2)


def kernel(continuous, categorical_indices, tables, W0, b0, W1, b1, W2, b2):
    cat_fmajor = categorical_indices.T.reshape(F * B)
    emb3 = _sc_gather(tables, cat_fmajor).reshape(NCHUNK, B, 128)
    W0e = W0[: F * D]
    W0c = W0[F * D:]
    W0p = jnp.concatenate(
        [W0e, jnp.zeros((NCHUNK * 128 - F * D, W0.shape[1]), W0.dtype)]
    ).reshape(NCHUNK, 128, W0.shape[1])
    return _mlp(emb3, continuous, W0p, W0c, b0[None, :], W1, b1[None, :],
                W2, b2[None, :])


# speedup vs baseline: 1.1158x; 1.0088x over previous
import functools

import jax
import jax.numpy as jnp
from jax import lax
from jax.experimental import pallas as pl
from jax.experimental.pallas import tpu as pltpu
from jax.experimental.pallas import tpu_sc as plsc

B = 16384
F = 26
V = 100000
D = 32
C = 16

NCHUNK = 7
NWORK = 32
PER_WORKER = B * F // NWORK
WLOOK = 512
NWIN = PER_WORKER // WLOOK
NG = WLOOK // 16
NSTREAM = WLOOK // 128

MLP_BLOCK = 1024


def _sc_gather(t2, cat2):
    mesh = plsc.VectorSubcoreMesh(core_axis_name="c", subcore_axis_name="s")
    cp = pltpu.CompilerParams(needs_layout_passes=False,
                              use_tc_tiling_on_sc=False)

    @functools.partial(
        pl.kernel,
        mesh=mesh,
        compiler_params=cp,
        out_type=jax.ShapeDtypeStruct((NCHUNK * B, 128), jnp.float32),
        scratch_types=[
            pltpu.VMEM((WLOOK, F), jnp.int32),
            pltpu.VMEM((WLOOK,), jnp.int32),
            pltpu.VMEM((WLOOK, D), jnp.float32),
            pltpu.SemaphoreType.DMA,
        ],
    )
    def gather_kernel(t2_hbm, cat_hbm, out_hbm, catblk_v, tidx_v, rows_v,
                      sem):
        wid = lax.axis_index("c") * 16 + lax.axis_index("s")
        b0 = wid * WLOOK
        pltpu.sync_copy(cat_hbm.at[pl.ds(b0, WLOOK)], catblk_v)

        @pl.loop(0, F)
        def _field(f):
            off = f * V
            f16 = jnp.full((16,), 0, jnp.int32) + f

            @pl.loop(0, NG)
            def _idx(g):
                row16 = lax.iota(jnp.int32, 16) + g * 16
                tidx_v[pl.ds(g * 16, 16)] = (
                    plsc.load_gather(catblk_v, [row16, f16]) + off)

            copies = [
                pltpu.async_copy(
                    t2_hbm.at[tidx_v.at[pl.ds(k * 128, 128)]],
                    rows_v.at[pl.ds(k * 128, 128)], sem)
                for k in range(NSTREAM)
            ]
            for c in copies:
                c.wait()

            j = lax.shift_right_logical(f, 2)
            l = lax.bitwise_and(f, 3) * D
            pltpu.sync_copy(
                rows_v,
                out_hbm.at[pl.ds(j * B + b0, WLOOK), pl.ds(l, D)])

            @pl.when(f >= F - 2)
            def _dup():
                pltpu.sync_copy(
                    rows_v,
                    out_hbm.at[pl.ds(j * B + b0, WLOOK), pl.ds(l + 64, D)])

    return gather_kernel(t2, cat2)


def _mlp_kernel(emb_ref, cont_ref, w0p_ref, w0c_ref, b0_ref, w1_ref, b1_ref,
                w2_ref, b2_ref, out_ref):
    x = jnp.dot(cont_ref[...], w0c_ref[...], preferred_element_type=jnp.float32)
    for j in range(NCHUNK):
        x = x + jnp.dot(emb_ref[j], w0p_ref[j],
                        preferred_element_type=jnp.float32)
    x = jnp.maximum(x + b0_ref[...], 0.0)
    x = jnp.maximum(jnp.dot(x, w1_ref[...], preferred_element_type=jnp.float32)
                    + b1_ref[...], 0.0)
    x = jnp.maximum(jnp.dot(x, w2_ref[...], preferred_element_type=jnp.float32)
                    + b2_ref[...], 0.0)
    out_ref[...] = x


def _mlp(emb3, cont, W0p, W0c, b0, W1, b1, W2, b2):
    grid = (B // MLP_BLOCK,)
    full = lambda shape: pl.BlockSpec(shape, lambda i: tuple(0 for _ in shape))
    return pl.pallas_call(
        _mlp_kernel,
        grid=grid,
        in_specs=[
            pl.BlockSpec((NCHUNK, MLP_BLOCK, 128), lambda i: (0, i, 0)),
            pl.BlockSpec((MLP_BLOCK, C), lambda i: (i, 0)),
            full(W0p.shape), full(W0c.shape), full(b0.shape),
            full(W1.shape), full(b1.shape), full(W2.shape), full(b2.shape),
        ],
        out_specs=pl.BlockSpec((MLP_BLOCK, W2.shape[1]), lambda i: (i, 0)),
        out_shape=jax.ShapeDtypeStruct((B, W2.shape[1]), jnp.float32),
    )(emb3, cont, W0p, W0c, b0, W1, b1, W2, b2)


def kernel(continuous, categorical_indices, tables, W0, b0, W1, b1, W2, b2):
    emb3 = _sc_gather(tables, categorical_indices).reshape(NCHUNK, B, 128)
    W0e = W0[: F * D]
    W0c = W0[F * D:]
    W0p = jnp.concatenate(
        [W0e, jnp.zeros((NCHUNK * 128 - F * D, W0.shape[1]), W0.dtype)]
    ).reshape(NCHUNK, 128, W0.shape[1])
    return _mlp(emb3, continuous, W0p, W0c, b0[None, :], W1, b1[None, :],
                W2, b2[None, :])
